# Initial kernel scaffold; baseline (speedup 1.0000x reference)
import jax
import jax.numpy as jnp
from jax.experimental import pallas as pl


def kernel(session_item, session_len, D, A, reversed_sess_item, mask, adj_indices, adj_values, embedding, pos_embedding, w1_w, w1_b, w2, glu1_w, glu1_b, glu2_w):
    raise NotImplementedError("write your pallas kernel here")


# speedup vs baseline: 1.0136x; 1.0136x over previous
import functools
import jax
import jax.numpy as jnp
from jax import lax
from jax.experimental import pallas as pl
from jax.experimental.pallas import tpu as pltpu
from jax.experimental.pallas import tpu_sc as plsc

N_NODE = 50000
E = 800000
EMB = 100
EMBP = 128
B = 1024
L = 50
LAYERS = 3
BETA = 0.01

NC = 2
NS = 16
NW = NC * NS

CH = 12544
TRASH = CH
ACC_ROWS = CH + 128


def _make_sc_gather(n_idx, n_rows, be):
    per_w = n_idx // NW
    nblk = per_w // be
    mesh = plsc.VectorSubcoreMesh(core_axis_name="c", subcore_axis_name="s")

    @functools.partial(
        pl.kernel, mesh=mesh,
        out_type=jax.ShapeDtypeStruct((n_idx, EMBP), jnp.float32),
        scratch_types=[
            pltpu.VMEM((be,), jnp.int32),
            pltpu.VMEM((be, EMBP), jnp.float32),
            pltpu.SemaphoreType.DMA,
        ],
    )
    def k(table_hbm, idx_hbm, out_hbm, idx_v, rows_v, sem):
        wid = lax.axis_index("s") * NC + lax.axis_index("c")
        w_base = wid * per_w

        def blk(i, carry):
            base = pl.multiple_of(w_base + i * be, 8)
            pltpu.sync_copy(idx_hbm.at[pl.ds(base, be)], idx_v)
            pltpu.async_copy(table_hbm.at[idx_v], rows_v, sem).wait()
            pltpu.sync_copy(rows_v, out_hbm.at[pl.ds(base, be), :])
            return carry

        lax.fori_loop(0, nblk, blk, 0)

    return k


def _make_sc_scatter(be):
    per_s = E // NS
    nblk = per_s // be
    ngrp = be // 16
    zrows = ACC_ROWS // NS
    frows = CH // NS
    mesh = plsc.VectorSubcoreMesh(core_axis_name="c", subcore_axis_name="s")

    @functools.partial(
        pl.kernel, mesh=mesh,
        out_type=jax.ShapeDtypeStruct((4 * CH, EMBP), jnp.float32),
        scratch_types=[
            pltpu.VMEM((be,), jnp.int32),
            pltpu.VMEM((be,), jnp.int32),
            pltpu.VMEM((be, EMBP), jnp.float32),
            pltpu.VMEM_SHARED((ACC_ROWS, EMBP), jnp.float32),
        ],
    )
    def k(rows_hbm, idx_hbm, zeros_hbm, out_hbm, idx_v, lidx_v, rows_v, acc_sh):
        cid = lax.axis_index("c")
        sid = lax.axis_index("s")
        s_base = sid * per_s

        for j in range(2):
            lo = (2 * cid + j) * CH
            zoff = pl.multiple_of(sid * zrows, 8)
            pltpu.sync_copy(zeros_hbm.at[pl.ds(zoff, zrows)],
                            acc_sh.at[pl.ds(zoff, zrows)])
            plsc.subcore_barrier()

            def blk(i, carry):
                base = pl.multiple_of(s_base + i * be, 8)
                pltpu.sync_copy(idx_hbm.at[pl.ds(base, be)], idx_v)
                pltpu.sync_copy(rows_hbm.at[pl.ds(base, be), :], rows_v)
                for g in range(ngrp):
                    r = idx_v[pl.ds(g * 16, 16)]
                    loc = r - lo
                    ok = (loc >= 0) & (loc < CH)
                    lidx_v[pl.ds(g * 16, 16)] = jnp.where(ok, loc, TRASH)
                pltpu.sync_copy(rows_v, acc_sh.at[lidx_v], add=True)
                return carry

            lax.fori_loop(0, nblk, blk, 0)
            plsc.subcore_barrier()
            foff = pl.multiple_of(sid * frows, 8)
            ooff = pl.multiple_of(lo + sid * frows, 8)
            pltpu.sync_copy(acc_sh.at[pl.ds(foff, frows)],
                            out_hbm.at[pl.ds(ooff, frows)])
            plsc.subcore_barrier()

    return k


def _tc_scale(rows, vals):
    n = rows.shape[0]
    bn = 1000
    grid = n // bn

    def body(v_ref, x_ref, o_ref):
        o_ref[...] = x_ref[...] * v_ref[...]

    return pl.pallas_call(
        body,
        grid=(grid,),
        in_specs=[
            pl.BlockSpec((bn, 1), lambda i: (i, 0)),
            pl.BlockSpec((bn, EMBP), lambda i: (i, 0)),
        ],
        out_specs=pl.BlockSpec((bn, EMBP), lambda i: (i, 0)),
        out_shape=jax.ShapeDtypeStruct((n, EMBP), jnp.float32),
    )(vals.reshape(n, 1), rows)


def _tc_attention(seq_h, seq_h1, maskf, sess_len, pos50, w1a, w1b, w1_b,
                  glu1_wt, glu1_b, glu2_wt, w2t):
    bb = 128
    grid = B // bb

    def body(sh_ref, sh1_ref, m_ref, len_ref, pos_ref, w1a_ref, w1b_ref,
             b1_ref, g1_ref, g1b_ref, g2_ref, w2_ref, sess_ref, s1_ref):
        sh = sh_ref[...]
        ln = len_ref[...]
        hs = jnp.sum(sh, axis=1) / ln
        sh2 = sh.reshape(bb * L, EMB)
        pos_term = jnp.dot(pos_ref[...], w1a_ref[...],
                           preferred_element_type=jnp.float32)
        seq_term = jnp.dot(sh2, w1b_ref[...],
                           preferred_element_type=jnp.float32)
        nh = jnp.tanh(seq_term.reshape(bb, L, EMB) + pos_term[None, :, :]
                      + b1_ref[...][None, :, :]).reshape(bb * L, EMB)
        hs_rep = jnp.broadcast_to(hs[:, None, :], (bb, L, EMB)).reshape(bb * L, EMB)
        gl = jax.nn.sigmoid(
            jnp.dot(nh, g1_ref[...], preferred_element_type=jnp.float32)
            + g1b_ref[...]
            + jnp.dot(hs_rep, g2_ref[...], preferred_element_type=jnp.float32))
        beta = jnp.sum(gl * w2_ref[...], axis=-1, keepdims=True)
        beta = beta * m_ref[...]
        sess_ref[...] = jnp.sum((beta * sh2).reshape(bb, L, EMB), axis=1)
        s1_ref[...] = jnp.sum(sh1_ref[...], axis=1) / ln

    return pl.pallas_call(
        body,
        grid=(grid,),
        in_specs=[
            pl.BlockSpec((bb, L, EMB), lambda i: (i, 0, 0)),
            pl.BlockSpec((bb, L, EMB), lambda i: (i, 0, 0)),
            pl.BlockSpec((bb * L, 1), lambda i: (i, 0)),
            pl.BlockSpec((bb, 1), lambda i: (i, 0)),
            pl.BlockSpec((L, EMB), lambda i: (0, 0)),
            pl.BlockSpec((EMB, EMB), lambda i: (0, 0)),
            pl.BlockSpec((EMB, EMB), lambda i: (0, 0)),
            pl.BlockSpec((1, EMB), lambda i: (0, 0)),
            pl.BlockSpec((EMB, EMB), lambda i: (0, 0)),
            pl.BlockSpec((1, EMB), lambda i: (0, 0)),
            pl.BlockSpec((EMB, EMB), lambda i: (0, 0)),
            pl.BlockSpec((1, EMB), lambda i: (0, 0)),
        ],
        out_specs=[
            pl.BlockSpec((bb, EMB), lambda i: (i, 0)),
            pl.BlockSpec((bb, EMB), lambda i: (i, 0)),
        ],
        out_shape=[
            jax.ShapeDtypeStruct((B, EMB), jnp.float32),
            jax.ShapeDtypeStruct((B, EMB), jnp.float32),
        ],
    )(seq_h, seq_h1, maskf, sess_len, pos50, w1a, w1b, w1_b.reshape(1, EMB),
      glu1_wt, glu1_b.reshape(1, EMB), glu2_wt, w2t)


def _tc_lineconv_loss(D, A, s1, sess, corrupted):
    def body(d_ref, a_ref, s1_ref, se_ref, co_ref, loss_ref):
        DA = jnp.dot(d_ref[...], a_ref[...], preferred_element_type=jnp.float32)
        s = s1_ref[...]
        acc = s
        for _ in range(LAYERS):
            s = jnp.dot(DA, s, preferred_element_type=jnp.float32)
            acc = acc + s
        lg = acc / (LAYERS + 1.0)
        sess = se_ref[...]
        pos = jnp.sum(sess * lg, axis=1)
        neg = jnp.sum(lg * co_ref[...], axis=1)
        loss = jnp.sum(-jnp.log(1e-08 + jax.nn.sigmoid(pos))
                       - jnp.log(1e-08 + (1.0 - jax.nn.sigmoid(neg))))
        loss_ref[...] = loss.reshape(1, 1)

    return pl.pallas_call(
        body,
        out_shape=jax.ShapeDtypeStruct((1, 1), jnp.float32),
    )(D, A, s1, sess, corrupted)


def kernel(session_item, session_len, D, A, reversed_sess_item, mask,
           adj_indices, adj_values, embedding, pos_embedding, w1_w, w1_b, w2,
           glu1_w, glu1_b, glu2_w):
    row = adj_indices[0].astype(jnp.int32)
    col = adj_indices[1].astype(jnp.int32)

    emb_pad = jnp.pad(embedding, ((0, 0), (0, EMBP - EMB)))
    zeros_acc = jnp.zeros((ACC_ROWS, EMBP), jnp.float32)

    edge_gather = _make_sc_gather(E, N_NODE, 40)
    scatter = _make_sc_scatter(80)

    cur = emb_pad
    acc = emb_pad
    for _ in range(LAYERS):
        gathered = edge_gather(cur, col)
        scaled = _tc_scale(gathered, adj_values)
        summed = scatter(scaled, row, zeros_acc)
        cur = summed[:N_NODE]
        acc = acc + cur
    item_hg_pad = acc / (LAYERS + 1.0)
    item_embeddings_hg = item_hg_pad[:, :EMB]

    zrow = jnp.zeros((1, EMBP), jnp.float32)
    table_hg = jnp.concatenate([zrow, item_hg_pad], axis=0)
    table0 = jnp.concatenate([zrow, emb_pad], axis=0)
    seq_gather = _make_sc_gather(B * L, N_NODE + 1, 40)
    seq_h = seq_gather(table_hg, reversed_sess_item.reshape(-1).astype(jnp.int32))
    seq_h1 = seq_gather(table0, session_item.reshape(-1).astype(jnp.int32))
    seq_h = seq_h[:, :EMB].reshape(B, L, EMB)
    seq_h1 = seq_h1[:, :EMB].reshape(B, L, EMB)

    w1a = w1_w[:, :EMB]
    w1b = w1_w[:, EMB:]
    sess_emb_hgnn, s1 = _tc_attention(
        seq_h, seq_h1, mask.astype(jnp.float32).reshape(B * L, 1), session_len,
        pos_embedding[:L], w1a.T, w1b.T, w1_b, glu1_w.T, glu1_b, glu2_w.T,
        w2.reshape(1, EMB))

    kp = jax.random.key(42)
    perm_r = jax.random.permutation(jax.random.fold_in(kp, 0), B)
    perm_c = jax.random.permutation(jax.random.fold_in(kp, 1), EMB)
    corrupted = sess_emb_hgnn[perm_r][:, perm_c]
    loss = _tc_lineconv_loss(D, A, s1, sess_emb_hgnn, corrupted)

    return (item_embeddings_hg, sess_emb_hgnn, BETA * loss[0, 0])
